# baseline (speedup 1.0000x reference)
import functools

import jax
import jax.numpy as jnp
from jax import lax
from jax.experimental import pallas as pl
from jax.experimental.pallas import tpu as pltpu
from jax.experimental.pallas import tpu_sc as plsc

DIM = 128
NUM_CORES = 2
NUM_SUBCORES = 16
NW = NUM_CORES * NUM_SUBCORES

CHUNK = 32
GE = 1
GSZ = (GE + 1) * CHUNK
NBUF = 4
K = 2


@functools.partial(jax.jit, static_argnums=(2,))
def _gather_rows(ids_flat, table, n_rows):
    rows_per_w = n_rows // NW
    groups = rows_per_w // GSZ
    assert groups % NBUF == 0 and groups // NBUF >= 2
    mesh = plsc.VectorSubcoreMesh(core_axis_name="c", subcore_axis_name="s")

    @functools.partial(
        pl.kernel,
        mesh=mesh,
        out_type=jax.ShapeDtypeStruct((n_rows, DIM), jnp.float32),
        scratch_types=[
            pltpu.VMEM((rows_per_w,), jnp.int32),
            pltpu.VMEM((NBUF, GE * CHUNK, DIM), jnp.float32),
            pltpu.VMEM((NBUF, CHUNK, DIM), jnp.float32),
            pltpu.VMEM_SHARED((NUM_SUBCORES * NBUF, CHUNK, DIM), jnp.float32),
            pltpu.SemaphoreType.DMA((NBUF,)),
            pltpu.SemaphoreType.DMA((NBUF,)),
            pltpu.SemaphoreType.DMA((NBUF,)),
            pltpu.SemaphoreType.DMA((NBUF,)),
            pltpu.SemaphoreType.DMA((NBUF,)),
        ],
    )
    def body(ids_hbm, table_hbm, out_hbm, idx_v, rows_v, rows2_v, spm,
             gsem, ssem, gsem2, csem2, ssem2):
        wid = lax.axis_index("s") * NUM_CORES + lax.axis_index("c")
        sid = lax.axis_index("s")
        base = wid * rows_per_w
        pltpu.sync_copy(ids_hbm.at[pl.ds(base, rows_per_w)], idx_v)

        def gather_e(g, buf, j):
            return pltpu.make_async_copy(
                table_hbm.at[idx_v.at[pl.ds(g * GSZ + j * CHUNK, CHUNK)]],
                rows_v.at[buf, pl.ds(j * CHUNK, CHUNK)],
                gsem.at[buf],
            )

        def store_e(g, buf):
            return pltpu.make_async_copy(
                rows_v.at[buf],
                out_hbm.at[pl.ds(base + g * GSZ, GE * CHUNK)],
                ssem.at[buf],
            )

        def gather_o(g, buf):
            return pltpu.make_async_copy(
                table_hbm.at[idx_v.at[pl.ds(g * GSZ + GE * CHUNK, CHUNK)]],
                rows2_v.at[buf],
                gsem2.at[buf],
            )

        def copy_o(buf):
            return pltpu.make_async_copy(
                rows2_v.at[buf],
                spm.at[sid * NBUF + buf],
                csem2.at[buf],
            )

        def store_o(g, buf):
            return pltpu.make_async_copy(
                spm.at[sid * NBUF + buf],
                out_hbm.at[pl.ds(base + g * GSZ + GE * CHUNK, CHUNK)],
                ssem2.at[buf],
            )

        def launch(g, buf):
            for j in range(GE):
                gather_e(g, buf, j).start()
            gather_o(g, buf).start()

        def emit(g, b, do_gather, do_store_wait, do_prev):
            if do_gather:
                bk = (b + K) % NBUF
                if do_store_wait:
                    store_e(g + K - NBUF, bk).wait()
                    store_o(g + K - NBUF, bk).wait()
                launch(g + K, bk)
            for j in range(GE):
                gather_e(g, b, j).wait()
            store_e(g, b).start()
            gather_o(g, b).wait()
            copy_o(b).start()
            if do_prev:
                bp = (b - 1) % NBUF
                copy_o(bp).wait()
                store_o(g - 1, bp).start()

        for i in range(K):
            launch(i, i)
        for g in range(NBUF):
            emit(g, g, True, g + K - NBUF >= 0, g >= 1)

        def outer(g0, carry):
            for j in range(NBUF):
                emit(g0 * NBUF + j, j, True, True, True)
            return carry

        lax.fori_loop(1, groups // NBUF - 1, outer, 0)

        for j in range(NBUF):
            g = groups - NBUF + j
            emit(g, j, g + K < groups, True, True)
        copy_o(NBUF - 1).wait()
        store_o(groups - 1, NBUF - 1).start()
        for j in range(NBUF):
            store_e(groups - NBUF + j, j).wait()
            store_o(groups - NBUF + j, j).wait()

    return body(ids_flat, table)


def kernel(input_ids, embedding_weight):
    b, l = input_ids.shape
    n_rows = b * l
    out = _gather_rows(input_ids.reshape(n_rows), embedding_weight, n_rows)
    return out.reshape(b, l, DIM)


# speedup vs baseline: 1.0660x; 1.0660x over previous
import functools

import jax
import jax.numpy as jnp
from jax import lax
from jax.experimental import pallas as pl
from jax.experimental.pallas import tpu as pltpu
from jax.experimental.pallas import tpu_sc as plsc

DIM = 128
NUM_CORES = 2
NUM_SUBCORES = 16
NW = NUM_CORES * NUM_SUBCORES

CE = 96
CO = 32
GSZ = CE + CO
NBUF = 4
K = 2


@functools.partial(jax.jit, static_argnums=(2,))
def _gather_rows(ids_flat, table, n_rows):
    rows_per_w = n_rows // NW
    groups = rows_per_w // GSZ
    assert groups % NBUF == 0 and groups // NBUF >= 2
    mesh = plsc.VectorSubcoreMesh(core_axis_name="c", subcore_axis_name="s")

    @functools.partial(
        pl.kernel,
        mesh=mesh,
        out_type=jax.ShapeDtypeStruct((n_rows, DIM), jnp.float32),
        scratch_types=[
            pltpu.VMEM((rows_per_w,), jnp.int32),
            pltpu.VMEM((NBUF, CE, DIM), jnp.float32),
            pltpu.VMEM((NBUF, CO, DIM), jnp.float32),
            pltpu.VMEM_SHARED((NUM_SUBCORES * NBUF, CO, DIM), jnp.float32),
            pltpu.SemaphoreType.DMA((NBUF,)),
            pltpu.SemaphoreType.DMA((NBUF,)),
            pltpu.SemaphoreType.DMA((NBUF,)),
            pltpu.SemaphoreType.DMA((NBUF,)),
            pltpu.SemaphoreType.DMA((NBUF,)),
        ],
    )
    def body(ids_hbm, table_hbm, out_hbm, idx_v, rows_v, rows2_v, spm,
             gsem, ssem, gsem2, csem2, ssem2):
        wid = lax.axis_index("s") * NUM_CORES + lax.axis_index("c")
        sid = lax.axis_index("s")
        base = wid * rows_per_w
        pltpu.sync_copy(ids_hbm.at[pl.ds(base, rows_per_w)], idx_v)

        def gather_e(g, buf):
            return pltpu.make_async_copy(
                table_hbm.at[idx_v.at[pl.ds(g * GSZ, CE)]],
                rows_v.at[buf],
                gsem.at[buf],
            )

        def store_e(g, buf):
            return pltpu.make_async_copy(
                rows_v.at[buf],
                out_hbm.at[pl.ds(base + g * GSZ, CE)],
                ssem.at[buf],
            )

        def gather_o(g, buf):
            return pltpu.make_async_copy(
                table_hbm.at[idx_v.at[pl.ds(g * GSZ + CE, CO)]],
                rows2_v.at[buf],
                gsem2.at[buf],
            )

        def copy_o(buf):
            return pltpu.make_async_copy(
                rows2_v.at[buf],
                spm.at[sid * NBUF + buf],
                csem2.at[buf],
            )

        def store_o(g, buf):
            return pltpu.make_async_copy(
                spm.at[sid * NBUF + buf],
                out_hbm.at[pl.ds(base + g * GSZ + CE, CO)],
                ssem2.at[buf],
            )

        def launch(g, buf):
            gather_e(g, buf).start()
            gather_o(g, buf).start()

        def emit(g, b, do_gather, do_store_wait, do_prev):
            if do_gather:
                bk = (b + K) % NBUF
                if do_store_wait:
                    store_e(g + K - NBUF, bk).wait()
                    store_o(g + K - NBUF, bk).wait()
                launch(g + K, bk)
            gather_e(g, b).wait()
            store_e(g, b).start()
            gather_o(g, b).wait()
            copy_o(b).start()
            if do_prev:
                bp = (b - 1) % NBUF
                copy_o(bp).wait()
                store_o(g - 1, bp).start()

        for i in range(K):
            launch(i, i)
        for g in range(NBUF):
            emit(g, g, True, g + K - NBUF >= 0, g >= 1)

        def outer(g0, carry):
            for j in range(NBUF):
                emit(g0 * NBUF + j, j, True, True, True)
            return carry

        lax.fori_loop(1, groups // NBUF - 1, outer, 0)

        for j in range(NBUF):
            g = groups - NBUF + j
            emit(g, j, g + K < groups, True, True)
        copy_o(NBUF - 1).wait()
        store_o(groups - 1, NBUF - 1).start()
        for j in range(NBUF):
            store_e(groups - NBUF + j, j).wait()
            store_o(groups - NBUF + j, j).wait()

    return body(ids_flat, table)


def kernel(input_ids, embedding_weight):
    b, l = input_ids.shape
    n_rows = b * l
    out = _gather_rows(input_ids.reshape(n_rows), embedding_weight, n_rows)
    return out.reshape(b, l, DIM)


# speedup vs baseline: 1.0843x; 1.0172x over previous
import functools

import jax
import jax.numpy as jnp
from jax import lax
from jax.experimental import pallas as pl
from jax.experimental.pallas import tpu as pltpu
from jax.experimental.pallas import tpu_sc as plsc

DIM = 128
NUM_CORES = 2
NUM_SUBCORES = 16
NW = NUM_CORES * NUM_SUBCORES

CE = 56
CO = 72
GSZ = CE + CO
NBUF = 4
K = 2


@functools.partial(jax.jit, static_argnums=(2,))
def _gather_rows(ids_flat, table, n_rows):
    rows_per_w = n_rows // NW
    groups = rows_per_w // GSZ
    assert groups % NBUF == 0 and groups // NBUF >= 2
    mesh = plsc.VectorSubcoreMesh(core_axis_name="c", subcore_axis_name="s")

    @functools.partial(
        pl.kernel,
        mesh=mesh,
        out_type=jax.ShapeDtypeStruct((n_rows, DIM), jnp.float32),
        scratch_types=[
            pltpu.VMEM((rows_per_w,), jnp.int32),
            pltpu.VMEM((NBUF, CE, DIM), jnp.float32),
            pltpu.VMEM((NBUF, CO, DIM), jnp.float32),
            pltpu.VMEM_SHARED((NUM_SUBCORES * NBUF, CO, DIM), jnp.float32),
            pltpu.SemaphoreType.DMA((NBUF,)),
            pltpu.SemaphoreType.DMA((NBUF,)),
            pltpu.SemaphoreType.DMA((NBUF,)),
            pltpu.SemaphoreType.DMA((NBUF,)),
            pltpu.SemaphoreType.DMA((NBUF,)),
        ],
    )
    def body(ids_hbm, table_hbm, out_hbm, idx_v, rows_v, rows2_v, spm,
             gsem, ssem, gsem2, csem2, ssem2):
        wid = lax.axis_index("s") * NUM_CORES + lax.axis_index("c")
        sid = lax.axis_index("s")
        base = wid * rows_per_w
        pltpu.sync_copy(ids_hbm.at[pl.ds(base, rows_per_w)], idx_v)

        def gather_e(g, buf):
            return pltpu.make_async_copy(
                table_hbm.at[idx_v.at[pl.ds(g * GSZ, CE)]],
                rows_v.at[buf],
                gsem.at[buf],
            )

        def store_e(g, buf):
            return pltpu.make_async_copy(
                rows_v.at[buf],
                out_hbm.at[pl.ds(base + g * GSZ, CE)],
                ssem.at[buf],
            )

        def gather_o(g, buf):
            return pltpu.make_async_copy(
                table_hbm.at[idx_v.at[pl.ds(g * GSZ + CE, CO)]],
                rows2_v.at[buf],
                gsem2.at[buf],
            )

        def copy_o(buf):
            return pltpu.make_async_copy(
                rows2_v.at[buf],
                spm.at[sid * NBUF + buf],
                csem2.at[buf],
            )

        def store_o(g, buf):
            return pltpu.make_async_copy(
                spm.at[sid * NBUF + buf],
                out_hbm.at[pl.ds(base + g * GSZ + CE, CO)],
                ssem2.at[buf],
            )

        def launch(g, buf):
            gather_e(g, buf).start()
            gather_o(g, buf).start()

        def emit(g, b, do_gather, do_store_wait, do_prev):
            if do_gather:
                bk = (b + K) % NBUF
                if do_store_wait:
                    store_e(g + K - NBUF, bk).wait()
                    store_o(g + K - NBUF, bk).wait()
                launch(g + K, bk)
            gather_e(g, b).wait()
            store_e(g, b).start()
            gather_o(g, b).wait()
            copy_o(b).start()
            if do_prev:
                bp = (b - 1) % NBUF
                copy_o(bp).wait()
                store_o(g - 1, bp).start()

        for i in range(K):
            launch(i, i)
        for g in range(NBUF):
            emit(g, g, True, g + K - NBUF >= 0, g >= 1)

        def outer(g0, carry):
            for j in range(NBUF):
                emit(g0 * NBUF + j, j, True, True, True)
            return carry

        lax.fori_loop(1, groups // NBUF - 1, outer, 0)

        for j in range(NBUF):
            g = groups - NBUF + j
            emit(g, j, g + K < groups, True, True)
        copy_o(NBUF - 1).wait()
        store_o(groups - 1, NBUF - 1).start()
        for j in range(NBUF):
            store_e(groups - NBUF + j, j).wait()
            store_o(groups - NBUF + j, j).wait()

    return body(ids_flat, table)


def kernel(input_ids, embedding_weight):
    b, l = input_ids.shape
    n_rows = b * l
    out = _gather_rows(input_ids.reshape(n_rows), embedding_weight, n_rows)
    return out.reshape(b, l, DIM)
